# Initial kernel scaffold; baseline (speedup 1.0000x reference)
import jax
import jax.numpy as jnp
from jax.experimental import pallas as pl


def kernel(target, context, negatives, syn0, syn1):
    raise NotImplementedError("write your pallas kernel here")


# speedup vs baseline: 1.8602x; 1.8602x over previous
import functools

import jax
import jax.numpy as jnp
from jax import lax
from jax.experimental import pallas as pl
from jax.experimental.pallas import tpu as pltpu
from jax.experimental.pallas import tpu_sc as plsc

B = 16384
L = 20
NNEG = 5
D = 64
NC, NS, LANES = 2, 16, 16
NW = NC * NS
EPW = B // NW
CB = 32
NBLK = EPW // CB
TN = 1 + NNEG
SLOTS = 16
PAD_SCORE = 1e4


def _sc_scores(ctx_flat, ctx32_flat, tn_flat, syn0, syn1):
    mesh = plsc.VectorSubcoreMesh(core_axis_name="c", subcore_axis_name="s")

    @functools.partial(
        pl.kernel,
        out_type=jax.ShapeDtypeStruct((B * SLOTS,), jnp.float32),
        mesh=mesh,
        compiler_params=pltpu.CompilerParams(
            needs_layout_passes=False, use_tc_tiling_on_sc=False),
        scratch_types=[
            pltpu.VMEM((CB * L,), jnp.int32),
            pltpu.VMEM((CB * 2 * LANES,), jnp.int32),
            pltpu.VMEM((CB * TN,), jnp.int32),
            pltpu.VMEM((CB * L, D), jnp.float32),
            pltpu.VMEM((CB * TN, D), jnp.float32),
            pltpu.VMEM((CB * SLOTS,), jnp.float32),
            pltpu.SemaphoreType.DMA,
        ],
    )
    def k(ctx_hbm, ctx32_hbm, tn_hbm, syn0_hbm, syn1_hbm, out_hbm,
          idx_ctx, idx32, idx_tn, rows_ctx, rows_tn, scores, sem):
        wid = lax.axis_index("s") * NC + lax.axis_index("c")
        lane = lax.iota(jnp.int32, LANES)

        def block(g, carry):
            base = wid * EPW + g * CB
            pltpu.sync_copy(ctx_hbm.at[pl.ds(base * L, CB * L)], idx_ctx)
            pltpu.sync_copy(ctx32_hbm.at[pl.ds(base * 2 * LANES, CB * 2 * LANES)], idx32)
            pltpu.sync_copy(tn_hbm.at[pl.ds(base * TN, CB * TN)], idx_tn)
            handles = []
            for t in range(CB * L // 128):
                handles.append(pltpu.async_copy(
                    syn0_hbm.at[idx_ctx.at[pl.ds(t * 128, 128)]],
                    rows_ctx.at[pl.ds(t * 128, 128)], sem))
            for t in range(2):
                half = CB * TN // 2
                handles.append(pltpu.async_copy(
                    syn1_hbm.at[idx_tn.at[pl.ds(t * half, half)]],
                    rows_tn.at[pl.ds(t * half, half)], sem))
            for h in handles:
                h.wait()

            def elem(e, carry2):
                v1 = idx32[pl.ds(e * 2 * LANES, LANES)]
                v2 = idx32[pl.ds(e * 2 * LANES + LANES, LANES)]
                cnt = (jnp.sum(jnp.where(v1 != 0, 1.0, 0.0))
                       + jnp.sum(jnp.where(v2 != 0, 1.0, 0.0)))
                rcp = 1.0 / jnp.full((LANES,), cnt, jnp.float32)
                mean = []
                for c in range(4):
                    acc = rows_ctx[e * L, pl.ds(c * LANES, LANES)]
                    for r in range(1, L):
                        acc = acc + rows_ctx[e * L + r, pl.ds(c * LANES, LANES)]
                    mean.append(acc * rcp)
                pacc = mean[0] * rows_tn[e * TN, pl.ds(0, LANES)]
                for c in range(1, 4):
                    pacc = pacc + mean[c] * rows_tn[e * TN, pl.ds(c * LANES, LANES)]
                s = jnp.full((LANES,), PAD_SCORE, jnp.float32)
                s = jnp.where(lane == 0, jnp.sum(pacc), s)
                for n in range(NNEG):
                    nacc = mean[0] * rows_tn[e * TN + 1 + n, pl.ds(0, LANES)]
                    for c in range(1, 4):
                        nacc = nacc + mean[c] * rows_tn[e * TN + 1 + n, pl.ds(c * LANES, LANES)]
                    s = jnp.where(lane == 1 + n, -jnp.sum(nacc), s)
                scores[pl.ds(e * SLOTS, SLOTS)] = s
                return carry2

            lax.fori_loop(0, CB, elem, 0)
            pltpu.sync_copy(scores, out_hbm.at[pl.ds(base * SLOTS, CB * SLOTS)])
            return carry

        lax.fori_loop(0, NBLK, block, 0)

    return k(ctx_flat, ctx32_flat, tn_flat, syn0, syn1)


def _tc_loss(scores2d):
    def body(s_ref, o_ref):
        x = s_ref[...]
        ls = jnp.minimum(x, 0.0) - jnp.log1p(jnp.exp(-jnp.abs(x)))
        o_ref[...] = jnp.full((1, 1), -jnp.sum(ls), jnp.float32)

    out = pl.pallas_call(
        body,
        out_shape=jax.ShapeDtypeStruct((1, 1), jnp.float32),
    )(scores2d)
    return out[0, 0]


def kernel(target, context, negatives, syn0, syn1):
    ctx_flat = context.reshape(-1).astype(jnp.int32)
    ctx32 = jnp.pad(context.astype(jnp.int32), ((0, 0), (0, 2 * LANES - L)))
    tn = jnp.concatenate([target[:, None].astype(jnp.int32),
                          negatives.astype(jnp.int32)], axis=1)
    scores = _sc_scores(ctx_flat, ctx32.reshape(-1), tn.reshape(-1), syn0, syn1)
    return _tc_loss(scores.reshape(B * SLOTS // 128, 128))
